# baseline (speedup 1.0000x reference)
import functools

import jax
import jax.numpy as jnp
from jax import lax
from jax.experimental import pallas as pl
from jax.experimental.pallas import tpu as pltpu
from jax.experimental.pallas import tpu_sc as plsc

_V = 1000
_L = 128
_CT = 8


_RB = 200


def _table_body(w_ref, x_ref, y_ref, d_ref, fidx_ref):
    w = w_ref[...]
    m = jnp.max(w, axis=1)
    s = jnp.sum(jnp.exp(w - m[:, None]), axis=1)
    lse = m + jnp.log(s)
    lse_bc = jnp.broadcast_to(lse[:, None], (_RB, _L))
    for ct in range(_CT - 1):
        d_ref[ct, :, :] = lse_bc - w[:, ct * _L:(ct + 1) * _L]
    wlast = jnp.concatenate(
        [w[:, (_CT - 1) * _L:],
         jnp.zeros((_RB, _CT * _L - _V), jnp.float32)], axis=1)
    d_ref[_CT - 1, :, :] = lse_bc - wlast

    @pl.when(pl.program_id(0) == 0)
    def _():
        x = x_ref[...]
        y = y_ref[...]
        fidx = (y >> 7) * (_V * _L) + x * _L + (y & (_L - 1))
        fidx_ref[...] = fidx.reshape(fidx_ref.shape)


def _tc_stage(w, x, y):
    nb = _V // _RB
    tok = x.shape[0] * x.shape[1]
    return pl.pallas_call(
        _table_body,
        grid=(nb,),
        in_specs=[
            pl.BlockSpec((_RB, _V), lambda i: (i, 0)),
            pl.BlockSpec(x.shape, lambda i: (0, 0)),
            pl.BlockSpec(x.shape, lambda i: (0, 0)),
        ],
        out_specs=(
            pl.BlockSpec((_CT, _RB, _L), lambda i: (0, i, 0)),
            pl.BlockSpec((tok,), lambda i: (0,)),
        ),
        out_shape=(
            jax.ShapeDtypeStruct((_CT, _V, _L), jnp.float32),
            jax.ShapeDtypeStruct((tok,), jnp.int32),
        ),
    )(w, x, y)


def _make_sc_gather(tok, nc, ns):
    nw = nc * ns
    per_w = tok // nw
    assert tok % nw == 0 and per_w % 16 == 0
    chunks = []
    off = 0
    while off < per_w:
        c = min(128, per_w - off)
        chunks.append((off, c))
        off += c
    mesh = plsc.VectorSubcoreMesh(core_axis_name="c", subcore_axis_name="s")

    @functools.partial(
        pl.kernel,
        out_type=jax.ShapeDtypeStruct((tok,), jnp.float32),
        mesh=mesh,
        scratch_types=[
            pltpu.VMEM((per_w,), jnp.int32),
            pltpu.VMEM((per_w,), jnp.float32),
            pltpu.SemaphoreType.DMA,
        ],
    )
    def sc_kernel(fidx_hbm, d_hbm, out_hbm, fidx, outv, sem):
        wid = lax.axis_index("s") * nc + lax.axis_index("c")
        base = wid * per_w
        pltpu.sync_copy(fidx_hbm.at[pl.ds(base, per_w)], fidx)
        copies = []
        for off, c in chunks:
            sl = pl.ds(off, c)
            copies.append(pltpu.async_copy(d_hbm.at[fidx.at[sl]], outv.at[sl], sem))
        for cp in copies:
            cp.wait()
        pltpu.sync_copy(outv, out_hbm.at[pl.ds(base, per_w)])

    return sc_kernel


def kernel(x, y, w_embed):
    b, t = x.shape
    tok = b * t
    info = plsc.get_sparse_core_info()
    d_tab, fidx = _tc_stage(w_embed, jnp.swapaxes(x, 0, 1),
                            jnp.swapaxes(y, 0, 1))
    sc = _make_sc_gather(tok, info.num_cores, info.num_subcores)
    loss = sc(fidx, d_tab.reshape(-1))
    return jnp.swapaxes(loss.reshape(t, b), 0, 1)


# speedup vs baseline: 1.0760x; 1.0760x over previous
import functools

import jax
import jax.numpy as jnp
from jax import lax
from jax.experimental import pallas as pl
from jax.experimental.pallas import tpu as pltpu
from jax.experimental.pallas import tpu_sc as plsc

_V = 1000
_L = 128
_CT = 8


def _table_body(w_ref, x_ref, y_ref, d_ref, fidx_ref):
    w = w_ref[...]
    m = jnp.max(w, axis=1)
    s = jnp.sum(jnp.exp(w - m[:, None]), axis=1)
    lse = m + jnp.log(s)
    lse_bc = jnp.broadcast_to(lse[:, None], (_V, _L))
    for ct in range(_CT - 1):
        d_ref[pl.ds(ct * _V, _V), :] = lse_bc - w[:, ct * _L:(ct + 1) * _L]
    wlast = jnp.concatenate(
        [w[:, (_CT - 1) * _L:],
         jnp.zeros((_V, _CT * _L - _V), jnp.float32)], axis=1)
    d_ref[pl.ds((_CT - 1) * _V, _V), :] = lse_bc - wlast
    x = x_ref[...]
    y = y_ref[...]
    fidx = (y >> 7) * (_V * _L) + x * _L + (y & (_L - 1))
    fidx_ref[...] = fidx.reshape(fidx_ref.shape)


def _tc_stage(w, x, y):
    return pl.pallas_call(
        _table_body,
        out_shape=(
            jax.ShapeDtypeStruct((_CT * _V, _L), jnp.float32),
            jax.ShapeDtypeStruct((x.shape[0] * x.shape[1],), jnp.int32),
        ),
    )(w, x, y)


def _make_sc_gather(tok, nc, ns):
    nw = nc * ns
    per_w = tok // nw
    assert tok % nw == 0 and per_w % 16 == 0
    chunks = [(0, per_w)]
    mesh = plsc.VectorSubcoreMesh(core_axis_name="c", subcore_axis_name="s")

    @functools.partial(
        pl.kernel,
        out_type=jax.ShapeDtypeStruct((tok,), jnp.float32),
        mesh=mesh,
        scratch_types=[
            pltpu.VMEM((per_w,), jnp.int32),
            pltpu.VMEM((per_w,), jnp.float32),
            pltpu.SemaphoreType.DMA,
        ],
    )
    def sc_kernel(fidx_hbm, d_hbm, out_hbm, fidx, outv, sem):
        wid = lax.axis_index("s") * nc + lax.axis_index("c")
        base = wid * per_w
        pltpu.sync_copy(fidx_hbm.at[pl.ds(base, per_w)], fidx)
        copies = []
        for off, c in chunks:
            sl = pl.ds(off, c)
            copies.append(pltpu.async_copy(d_hbm.at[fidx.at[sl]], outv.at[sl], sem))
        for cp in copies:
            cp.wait()
        pltpu.sync_copy(outv, out_hbm.at[pl.ds(base, per_w)])

    return sc_kernel


def kernel(x, y, w_embed):
    b, t = x.shape
    tok = b * t
    info = plsc.get_sparse_core_info()
    d_tab, fidx = _tc_stage(w_embed, jnp.swapaxes(x, 0, 1),
                            jnp.swapaxes(y, 0, 1))
    sc = _make_sc_gather(tok, info.num_cores, info.num_subcores)
    loss = sc(fidx, d_tab.reshape(-1))
    return jnp.swapaxes(loss.reshape(t, b), 0, 1)
